# baseline (speedup 1.0000x reference)
import jax
import jax.numpy as jnp
from jax import lax
from jax.experimental import pallas as pl
from jax.experimental.pallas import tpu as pltpu
from jax.experimental.pallas import tpu_sc as plsc

B, C, D = 16384, 10, 64
NC, NS = 2, 16
NW = NC * NS
ROWS_W = B // NW
CHUNK = 128
NCHUNK = ROWS_W // CHUNK
IDXW = 128
G = CHUNK * C // IDXW


def _body(idx_hbm, table_hbm, out_hbm, idx_v, rows_v, out_v, sem):
    w = lax.axis_index("s") * NC + lax.axis_index("c")
    for i in range(NCHUNK):
        pltpu.sync_copy(idx_hbm.at[w, i], idx_v)
        copies = [
            pltpu.async_copy(
                table_hbm.at[idx_v.at[j]],
                rows_v.at[pl.ds(j * IDXW, IDXW)],
                sem,
            )
            for j in range(G)
        ]
        for cp in copies:
            cp.wait()

        def acc_row(b, carry):
            for j in range(D // 16):
                s = rows_v[b * C, pl.ds(j * 16, 16)]
                for c in range(1, C):
                    s = s + rows_v[b * C + c, pl.ds(j * 16, 16)]
                out_v[b, pl.ds(j * 16, 16)] = s * (1.0 / C)
            return carry

        lax.fori_loop(0, CHUNK, acc_row, 0)
        base = (w * NCHUNK + i) * CHUNK
        pltpu.sync_copy(out_v, out_hbm.at[pl.ds(base, CHUNK)])


@jax.jit
def kernel(context_words, input_embeddings):
    idx = context_words.astype(jnp.int32).reshape(NW, NCHUNK, G, IDXW)
    f = pl.kernel(
        _body,
        out_type=jax.ShapeDtypeStruct((B, D), jnp.float32),
        mesh=plsc.VectorSubcoreMesh(core_axis_name="c", subcore_axis_name="s"),
        scratch_types=[
            pltpu.VMEM((G, IDXW), jnp.int32),
            pltpu.VMEM((CHUNK * C, D), jnp.float32),
            pltpu.VMEM((CHUNK, D), jnp.float32),
            pltpu.SemaphoreType.DMA,
        ],
        compiler_params=pltpu.CompilerParams(use_tc_tiling_on_sc=False),
    )
    return f(idx, input_embeddings)


# speedup vs baseline: 1.0311x; 1.0311x over previous
import jax
import jax.numpy as jnp
from jax import lax
from jax.experimental import pallas as pl
from jax.experimental.pallas import tpu as pltpu
from jax.experimental.pallas import tpu_sc as plsc

B, C, D = 16384, 10, 64
NC, NS = 2, 16
NW = NC * NS
ROWS_W = B // NW
CHUNK = 128
NCHUNK = ROWS_W // CHUNK
IDXW = 128
G = CHUNK * C // IDXW


def _body(idx_hbm, table_hbm, out_hbm, idx_v, rows_v, out_v, sem):
    w = lax.axis_index("s") * NC + lax.axis_index("c")
    for i in range(NCHUNK):
        pltpu.sync_copy(idx_hbm.at[w, i], idx_v)
        copies = [
            pltpu.async_copy(table_hbm.at[idx_v.at[c]], rows_v.at[c], sem)
            for c in range(C)
        ]
        for cp in copies:
            cp.wait()

        @plsc.parallel_loop(0, CHUNK, step=1, unroll=2)
        def acc_row(b):
            for j in range(D // 16):
                s = rows_v[0, b, pl.ds(j * 16, 16)]
                for c in range(1, C):
                    s = s + rows_v[c, b, pl.ds(j * 16, 16)]
                out_v[b, pl.ds(j * 16, 16)] = s * (1.0 / C)

        base = (w * NCHUNK + i) * CHUNK
        pltpu.sync_copy(out_v, out_hbm.at[pl.ds(base, CHUNK)])


@jax.jit
def kernel(context_words, input_embeddings):
    idx = context_words.astype(jnp.int32).reshape(NW, NCHUNK, CHUNK, C)
    idx = idx.transpose(0, 1, 3, 2)
    f = pl.kernel(
        _body,
        out_type=jax.ShapeDtypeStruct((B, D), jnp.float32),
        mesh=plsc.VectorSubcoreMesh(core_axis_name="c", subcore_axis_name="s"),
        scratch_types=[
            pltpu.VMEM((C, CHUNK), jnp.int32),
            pltpu.VMEM((C, CHUNK, D), jnp.float32),
            pltpu.VMEM((CHUNK, D), jnp.float32),
            pltpu.SemaphoreType.DMA,
        ],
        compiler_params=pltpu.CompilerParams(use_tc_tiling_on_sc=False),
    )
    return f(idx, input_embeddings)


# speedup vs baseline: 1.1198x; 1.0860x over previous
import jax
import jax.numpy as jnp
from jax import lax
from jax.experimental import pallas as pl
from jax.experimental.pallas import tpu as pltpu
from jax.experimental.pallas import tpu_sc as plsc

B, C, D = 16384, 10, 64
NC, NS = 2, 16
NW = NC * NS
ROWS_W = B // NW
CHUNK = 64
NCHUNK = ROWS_W // CHUNK
PL = 128


def _body(idx_hbm, table_hbm, out_hbm, idx_v, rows_v, out_v, sem):
    w = lax.axis_index("s") * NC + lax.axis_index("c")

    def chunk_body(i, carry):
        pltpu.sync_copy(idx_hbm.at[w, i], idx_v)
        copies = [
            pltpu.async_copy(table_hbm.at[idx_v.at[c]], rows_v.at[c], sem)
            for c in range(C)
        ]
        for cp in copies:
            cp.wait()

        @plsc.parallel_loop(0, CHUNK, step=1, unroll=2)
        def acc_row(b):
            for j in range(D // 16):
                s = rows_v[0, b, pl.ds(j * 16, 16)]
                for c in range(1, C):
                    s = s + rows_v[c, b, pl.ds(j * 16, 16)]
                out_v[b, pl.ds(j * 16, 16)] = s * (1.0 / C)

        base = (w * NCHUNK + i) * CHUNK
        pltpu.sync_copy(out_v, out_hbm.at[pl.ds(base, CHUNK)])
        return carry

    lax.fori_loop(0, NCHUNK, chunk_body, 0)


@jax.jit
def kernel(context_words, input_embeddings):
    idx = context_words.astype(jnp.int32).reshape(NW, NCHUNK, CHUNK, C)
    idx = idx.transpose(0, 1, 3, 2)
    padded = jnp.pad(input_embeddings, ((0, 0), (0, PL - D)))
    f = pl.kernel(
        _body,
        out_type=jax.ShapeDtypeStruct((B, D), jnp.float32),
        mesh=plsc.VectorSubcoreMesh(core_axis_name="c", subcore_axis_name="s"),
        scratch_types=[
            pltpu.VMEM((C, CHUNK), jnp.int32),
            pltpu.VMEM((C, CHUNK, PL), jnp.float32),
            pltpu.VMEM((CHUNK, D), jnp.float32),
            pltpu.SemaphoreType.DMA,
        ],
        compiler_params=pltpu.CompilerParams(use_tc_tiling_on_sc=True),
    )
    return f(idx, padded)
